# baseline (speedup 1.0000x reference)
import functools

import jax
import jax.numpy as jnp
from jax import lax
from jax.experimental import pallas as pl
from jax.experimental.pallas import tpu as pltpu
from jax.experimental.pallas import tpu_sc as plsc


def _transform_body(tbl_ref, w_ref, b_ref, out_ref):
    out_ref[...] = (
        jnp.dot(tbl_ref[...], w_ref[...], preferred_element_type=jnp.float32)
        + b_ref[...]
    )


@functools.partial(jax.jit, static_argnames=("blk",))
def _transform(table, W, b, blk=2000):
    V, D = table.shape
    return pl.pallas_call(
        _transform_body,
        grid=(V // blk,),
        in_specs=[
            pl.BlockSpec((blk, D), lambda i: (i, 0)),
            pl.BlockSpec((D, D), lambda i: (0, 0)),
            pl.BlockSpec((1, D), lambda i: (0, 0)),
        ],
        out_specs=pl.BlockSpec((blk, D), lambda i: (i, 0)),
        out_shape=jax.ShapeDtypeStruct((V, D), jnp.float32),
    )(table, W, b.reshape(1, D))


@functools.lru_cache(maxsize=None)
def _make_gather(batch, hist, D):
    info = plsc.get_sparse_core_info()
    nc, ns = info.num_cores, info.num_subcores
    nw = nc * ns
    b_per_w = batch // nw
    assert b_per_w * nw == batch

    mesh = plsc.VectorSubcoreMesh(core_axis_name="c", subcore_axis_name="s")

    @functools.partial(
        pl.kernel,
        out_type=jax.ShapeDtypeStruct((batch, hist, D), jnp.float32),
        mesh=mesh,
        compiler_params=pltpu.CompilerParams(
            use_tc_tiling_on_sc=True, needs_layout_passes=True),
        scratch_types=[
            pltpu.VMEM((b_per_w, hist), jnp.int32),
            pltpu.VMEM((hist, D), jnp.float32),
            pltpu.VMEM((hist, D), jnp.float32),
            pltpu.SemaphoreType.DMA,
            pltpu.SemaphoreType.DMA,
            pltpu.SemaphoreType.DMA,
            pltpu.SemaphoreType.DMA,
        ],
    )
    def gather_k(tbl_hbm, idx_hbm, out_hbm, idx_v, buf0, buf1,
                 gsem0, gsem1, wsem0, wsem1):
        wid = lax.axis_index("s") * nc + lax.axis_index("c")
        base = wid * b_per_w

        pltpu.sync_copy(idx_hbm.at[pl.ds(base, b_per_w)], idx_v)

        bufs = (buf0, buf1)
        gsems = (gsem0, gsem1)
        wsems = (wsem0, wsem1)

        def gather(k, p):
            return pltpu.make_async_copy(
                tbl_hbm.at[idx_v.at[k]], bufs[p], gsems[p])

        def writeback(k, p):
            return pltpu.make_async_copy(
                bufs[p], out_hbm.at[base + k], wsems[p])

        gather(0, 0).start()

        def body(g, _):
            for p in range(2):
                k = 2 * g + p

                @pl.when(k + 1 < b_per_w)
                def _():
                    @pl.when(k >= 1)
                    def _():
                        writeback(k - 1, 1 - p).wait()
                    gather(k + 1, 1 - p).start()

                gather(k, p).wait()
                writeback(k, p).start()
            return 0

        lax.fori_loop(0, b_per_w // 2, body, 0)

        writeback(b_per_w - 2, 0).wait()
        writeback(b_per_w - 1, 1).wait()

    return gather_k


N_SPLIT = 4


def kernel(utterances, table, W, b):
    batch, hist = utterances.shape
    D = table.shape[1]
    cb = batch // N_SPLIT

    transformed = _transform(table, W, b)
    idx = utterances.astype(jnp.int32)
    gather = _make_gather(cb, hist, D)
    outs = [gather(transformed, idx[c * cb:(c + 1) * cb]) for c in range(N_SPLIT)]
    return jnp.concatenate(outs, axis=0)


# speedup vs baseline: 2.5794x; 2.5794x over previous
import functools

import jax
import jax.numpy as jnp
from jax import lax
from jax.experimental import pallas as pl
from jax.experimental.pallas import tpu as pltpu
from jax.experimental.pallas import tpu_sc as plsc


def _transform_body(tbl_ref, w_ref, b_ref, out_ref):
    out_ref[...] = (
        jnp.dot(tbl_ref[...], w_ref[...], preferred_element_type=jnp.float32)
        + b_ref[...]
    )


@functools.partial(jax.jit, static_argnames=("blk",))
def _transform(table, W, b, blk=2000):
    V, D = table.shape
    return pl.pallas_call(
        _transform_body,
        grid=(V // blk,),
        in_specs=[
            pl.BlockSpec((blk, D), lambda i: (i, 0)),
            pl.BlockSpec((D, D), lambda i: (0, 0)),
            pl.BlockSpec((1, D), lambda i: (0, 0)),
        ],
        out_specs=pl.BlockSpec((blk, D), lambda i: (i, 0)),
        out_shape=jax.ShapeDtypeStruct((V, D), jnp.float32),
    )(table, W, b.reshape(1, D))


@functools.lru_cache(maxsize=None)
def _make_gather(batch, hist, D):
    info = plsc.get_sparse_core_info()
    nc, ns = info.num_cores, info.num_subcores
    nw = nc * ns
    b_per_w = batch // nw
    assert b_per_w * nw == batch and hist % 2 == 0

    mesh = plsc.VectorSubcoreMesh(core_axis_name="c", subcore_axis_name="s")

    @functools.partial(
        pl.kernel,
        out_type=jax.ShapeDtypeStruct((hist, batch, D), jnp.float32),
        mesh=mesh,
        scratch_types=[
            pltpu.VMEM((hist, b_per_w), jnp.int32),
            pltpu.VMEM((b_per_w, D), jnp.float32),
            pltpu.VMEM((b_per_w, D), jnp.float32),
            pltpu.SemaphoreType.DMA,
            pltpu.SemaphoreType.DMA,
            pltpu.SemaphoreType.DMA,
            pltpu.SemaphoreType.DMA,
        ],
    )
    def gather_k(tbl_hbm, idx_hbm, out_hbm, idx_v, buf0, buf1,
                 gsem0, gsem1, wsem0, wsem1):
        wid = lax.axis_index("s") * nc + lax.axis_index("c")
        base = wid * b_per_w

        pltpu.sync_copy(idx_hbm.at[:, pl.ds(base, b_per_w)], idx_v)

        bufs = (buf0, buf1)
        gsems = (gsem0, gsem1)
        wsems = (wsem0, wsem1)

        def gather(h, p):
            return pltpu.make_async_copy(
                tbl_hbm.at[idx_v.at[h]], bufs[p], gsems[p])

        def writeback(h, p):
            return pltpu.make_async_copy(
                bufs[p], out_hbm.at[h, pl.ds(base, b_per_w)], wsems[p])

        gather(0, 0).start()

        def body(g, _):
            for p in range(2):
                h = 2 * g + p

                @pl.when(h + 1 < hist)
                def _():
                    @pl.when(h >= 1)
                    def _():
                        writeback(h - 1, 1 - p).wait()
                    gather(h + 1, 1 - p).start()

                gather(h, p).wait()
                writeback(h, p).start()
            return 0

        lax.fori_loop(0, hist // 2, body, 0)

        writeback(hist - 2, 0).wait()
        writeback(hist - 1, 1).wait()

    return gather_k


def kernel(utterances, table, W, b):
    batch, hist = utterances.shape
    D = table.shape[1]

    transformed = _transform(table, W, b)
    idx_t = jnp.transpose(utterances.astype(jnp.int32))
    out_t = _make_gather(batch, hist, D)(transformed, idx_t)
    return jnp.transpose(out_t, (1, 0, 2))


# speedup vs baseline: 2.8953x; 1.1225x over previous
import functools

import jax
import jax.numpy as jnp
from jax import lax
from jax.experimental import pallas as pl
from jax.experimental.pallas import tpu as pltpu
from jax.experimental.pallas import tpu_sc as plsc


def _transform_body(tbl_ref, w_ref, b_ref, out_ref):
    out_ref[...] = (
        jnp.dot(tbl_ref[...], w_ref[...], preferred_element_type=jnp.float32)
        + b_ref[...]
    )


@functools.partial(jax.jit, static_argnames=("blk",))
def _transform(table, W, b, blk=5000):
    V, D = table.shape
    return pl.pallas_call(
        _transform_body,
        grid=(V // blk,),
        in_specs=[
            pl.BlockSpec((blk, D), lambda i: (i, 0)),
            pl.BlockSpec((D, D), lambda i: (0, 0)),
            pl.BlockSpec((1, D), lambda i: (0, 0)),
        ],
        out_specs=pl.BlockSpec((blk, D), lambda i: (i, 0)),
        out_shape=jax.ShapeDtypeStruct((V, D), jnp.float32),
    )(table, W, b.reshape(1, D))


@functools.lru_cache(maxsize=None)
def _make_gather(batch, hist, D):
    info = plsc.get_sparse_core_info()
    nc, ns = info.num_cores, info.num_subcores
    nw = nc * ns
    b_per_w = batch // nw
    assert b_per_w * nw == batch and hist % 2 == 0

    mesh = plsc.VectorSubcoreMesh(core_axis_name="c", subcore_axis_name="s")

    @functools.partial(
        pl.kernel,
        out_type=jax.ShapeDtypeStruct((hist, batch, D), jnp.float32),
        mesh=mesh,
        scratch_types=[
            pltpu.VMEM((hist, b_per_w), jnp.int32),
            pltpu.VMEM((b_per_w, D), jnp.float32),
            pltpu.VMEM((b_per_w, D), jnp.float32),
            pltpu.SemaphoreType.DMA,
            pltpu.SemaphoreType.DMA,
            pltpu.SemaphoreType.DMA,
            pltpu.SemaphoreType.DMA,
        ],
    )
    def gather_k(tbl_hbm, idx_hbm, out_hbm, idx_v, buf0, buf1,
                 gsem0, gsem1, wsem0, wsem1):
        wid = lax.axis_index("s") * nc + lax.axis_index("c")
        base = wid * b_per_w

        pltpu.sync_copy(idx_hbm.at[:, pl.ds(base, b_per_w)], idx_v)

        bufs = (buf0, buf1)
        gsems = (gsem0, gsem1)
        wsems = (wsem0, wsem1)

        def gather(h, p):
            return pltpu.make_async_copy(
                tbl_hbm.at[idx_v.at[h]], bufs[p], gsems[p])

        def writeback(h, p):
            return pltpu.make_async_copy(
                bufs[p], out_hbm.at[h, pl.ds(base, b_per_w)], wsems[p])

        gather(0, 0).start()

        def body(g, _):
            for p in range(2):
                h = 2 * g + p

                @pl.when(h + 1 < hist)
                def _():
                    @pl.when(h >= 1)
                    def _():
                        writeback(h - 1, 1 - p).wait()
                    gather(h + 1, 1 - p).start()

                gather(h, p).wait()
                writeback(h, p).start()
            return 0

        lax.fori_loop(0, hist // 2, body, 0)

        writeback(hist - 2, 0).wait()
        writeback(hist - 1, 1).wait()

    return gather_k


def kernel(utterances, table, W, b):
    batch, hist = utterances.shape
    D = table.shape[1]

    transformed = _transform(table, W, b)
    idx_t = jnp.transpose(utterances.astype(jnp.int32))
    out_t = _make_gather(batch, hist, D)(transformed, idx_t)
    return jnp.transpose(out_t, (1, 0, 2))


# speedup vs baseline: 3.0054x; 1.0380x over previous
import functools

import jax
import jax.numpy as jnp
from jax import lax
from jax.experimental import pallas as pl
from jax.experimental.pallas import tpu as pltpu
from jax.experimental.pallas import tpu_sc as plsc


def _transform_body(tbl_ref, w_ref, b_ref, out_ref):
    out_ref[...] = (
        jnp.dot(tbl_ref[...], w_ref[...], preferred_element_type=jnp.float32)
        + b_ref[...]
    )


@functools.partial(jax.jit, static_argnames=("blk",))
def _transform(table, W, b, blk=10000):
    V, D = table.shape
    return pl.pallas_call(
        _transform_body,
        grid=(V // blk,),
        in_specs=[
            pl.BlockSpec((blk, D), lambda i: (i, 0)),
            pl.BlockSpec((D, D), lambda i: (0, 0)),
            pl.BlockSpec((1, D), lambda i: (0, 0)),
        ],
        out_specs=pl.BlockSpec((blk, D), lambda i: (i, 0)),
        out_shape=jax.ShapeDtypeStruct((V, D), jnp.float32),
    )(table, W, b.reshape(1, D))


@functools.lru_cache(maxsize=None)
def _make_gather(batch, hist, D):
    info = plsc.get_sparse_core_info()
    nc, ns = info.num_cores, info.num_subcores
    nw = nc * ns
    b_per_w = batch // nw
    assert b_per_w * nw == batch and hist % 2 == 0

    mesh = plsc.VectorSubcoreMesh(core_axis_name="c", subcore_axis_name="s")

    @functools.partial(
        pl.kernel,
        out_type=jax.ShapeDtypeStruct((hist, batch, D), jnp.float32),
        mesh=mesh,
        scratch_types=[
            pltpu.VMEM((hist, b_per_w), jnp.int32),
            pltpu.VMEM((b_per_w, D), jnp.float32),
            pltpu.VMEM((b_per_w, D), jnp.float32),
            pltpu.SemaphoreType.DMA,
            pltpu.SemaphoreType.DMA,
            pltpu.SemaphoreType.DMA,
            pltpu.SemaphoreType.DMA,
        ],
    )
    def gather_k(tbl_hbm, idx_hbm, out_hbm, idx_v, buf0, buf1,
                 gsem0, gsem1, wsem0, wsem1):
        wid = lax.axis_index("s") * nc + lax.axis_index("c")
        base = wid * b_per_w

        pltpu.sync_copy(idx_hbm.at[:, pl.ds(base, b_per_w)], idx_v)

        bufs = (buf0, buf1)
        gsems = (gsem0, gsem1)
        wsems = (wsem0, wsem1)

        def gather(h, p):
            return pltpu.make_async_copy(
                tbl_hbm.at[idx_v.at[h]], bufs[p], gsems[p])

        def writeback(h, p):
            return pltpu.make_async_copy(
                bufs[p], out_hbm.at[h, pl.ds(base, b_per_w)], wsems[p])

        gather(0, 0).start()

        def body(g, _):
            for p in range(2):
                h = 2 * g + p

                @pl.when(h + 1 < hist)
                def _():
                    @pl.when(h >= 1)
                    def _():
                        writeback(h - 1, 1 - p).wait()
                    gather(h + 1, 1 - p).start()

                gather(h, p).wait()
                writeback(h, p).start()
            return 0

        lax.fori_loop(0, hist // 2, body, 0)

        writeback(hist - 2, 0).wait()
        writeback(hist - 1, 1).wait()

    return gather_k


def kernel(utterances, table, W, b):
    batch, hist = utterances.shape
    D = table.shape[1]

    transformed = _transform(table, W, b)
    idx_t = jnp.transpose(utterances.astype(jnp.int32))
    out_t = _make_gather(batch, hist, D)(transformed, idx_t)
    return jnp.transpose(out_t, (1, 0, 2))


# speedup vs baseline: 3.0411x; 1.0119x over previous
import functools

import jax
import jax.numpy as jnp
from jax import lax
from jax.experimental import pallas as pl
from jax.experimental.pallas import tpu as pltpu
from jax.experimental.pallas import tpu_sc as plsc


def _transform_body(tbl_ref, w_ref, b_ref, out_ref):
    out_ref[...] = (
        jnp.dot(tbl_ref[...], w_ref[...], preferred_element_type=jnp.float32)
        + b_ref[...]
    )


@functools.partial(jax.jit, static_argnames=("blk",))
def _transform(table, W, b, blk=20000):
    V, D = table.shape
    return pl.pallas_call(
        _transform_body,
        grid=(V // blk,),
        in_specs=[
            pl.BlockSpec((blk, D), lambda i: (i, 0)),
            pl.BlockSpec((D, D), lambda i: (0, 0)),
            pl.BlockSpec((1, D), lambda i: (0, 0)),
        ],
        out_specs=pl.BlockSpec((blk, D), lambda i: (i, 0)),
        out_shape=jax.ShapeDtypeStruct((V, D), jnp.float32),
    )(table, W, b.reshape(1, D))


@functools.lru_cache(maxsize=None)
def _make_gather(batch, hist, D):
    info = plsc.get_sparse_core_info()
    nc, ns = info.num_cores, info.num_subcores
    nw = nc * ns
    b_per_w = batch // nw
    assert b_per_w * nw == batch and hist % 2 == 0

    mesh = plsc.VectorSubcoreMesh(core_axis_name="c", subcore_axis_name="s")

    @functools.partial(
        pl.kernel,
        out_type=jax.ShapeDtypeStruct((hist, batch, D), jnp.float32),
        mesh=mesh,
        scratch_types=[
            pltpu.VMEM((hist, b_per_w), jnp.int32),
            pltpu.VMEM((b_per_w, D), jnp.float32),
            pltpu.VMEM((b_per_w, D), jnp.float32),
            pltpu.SemaphoreType.DMA,
            pltpu.SemaphoreType.DMA,
            pltpu.SemaphoreType.DMA,
            pltpu.SemaphoreType.DMA,
        ],
    )
    def gather_k(tbl_hbm, idx_hbm, out_hbm, idx_v, buf0, buf1,
                 gsem0, gsem1, wsem0, wsem1):
        wid = lax.axis_index("s") * nc + lax.axis_index("c")
        base = wid * b_per_w

        pltpu.sync_copy(idx_hbm.at[:, pl.ds(base, b_per_w)], idx_v)

        bufs = (buf0, buf1)
        gsems = (gsem0, gsem1)
        wsems = (wsem0, wsem1)

        def gather(h, p):
            return pltpu.make_async_copy(
                tbl_hbm.at[idx_v.at[h]], bufs[p], gsems[p])

        def writeback(h, p):
            return pltpu.make_async_copy(
                bufs[p], out_hbm.at[h, pl.ds(base, b_per_w)], wsems[p])

        gather(0, 0).start()

        def body(g, _):
            for p in range(2):
                h = 2 * g + p

                @pl.when(h + 1 < hist)
                def _():
                    @pl.when(h >= 1)
                    def _():
                        writeback(h - 1, 1 - p).wait()
                    gather(h + 1, 1 - p).start()

                gather(h, p).wait()
                writeback(h, p).start()
            return 0

        lax.fori_loop(0, hist // 2, body, 0)

        writeback(hist - 2, 0).wait()
        writeback(hist - 1, 1).wait()

    return gather_k


def kernel(utterances, table, W, b):
    batch, hist = utterances.shape
    D = table.shape[1]

    transformed = _transform(table, W, b)
    idx_t = jnp.transpose(utterances.astype(jnp.int32))
    out_t = _make_gather(batch, hist, D)(transformed, idx_t)
    return jnp.transpose(out_t, (1, 0, 2))
